# Initial kernel scaffold; baseline (speedup 1.0000x reference)
import jax
import jax.numpy as jnp
from jax.experimental import pallas as pl


def kernel(input_ids, embedding_matrix_1, embedding_matrix_2):
    raise NotImplementedError("write your pallas kernel here")


# speedup vs baseline: 1.1692x; 1.1692x over previous
import functools

import jax
import jax.numpy as jnp
from jax import lax
from jax.experimental import pallas as pl
from jax.experimental.pallas import tpu as pltpu
from jax.experimental.pallas import tpu_sc as plsc

NUM_EMB = 1000000
D = 64
B, L = 16384, 50
N = B * L

NC, NS = 2, 16
NW = NC * NS
PER_W = N // NW
CHUNK = 128
GROUP = 8
ROWS = CHUNK * GROUP
ROUNDS = PER_W // ROWS
N_CHUNKS = PER_W // CHUNK


def _sc_gather(idx, table):
    mesh = plsc.VectorSubcoreMesh(core_axis_name="c", subcore_axis_name="s")

    @functools.partial(
        pl.kernel,
        mesh=mesh,
        out_type=jax.ShapeDtypeStruct((N, D), jnp.float32),
        compiler_params=pltpu.CompilerParams(use_tc_tiling_on_sc=False),
        scratch_types=[
            pltpu.VMEM((N_CHUNKS, CHUNK), jnp.int32),
            pltpu.VMEM((ROWS, D), jnp.float32),
            pltpu.SemaphoreType.DMA,
        ],
    )
    def k(idx_hbm, table_hbm, out_hbm, idx_v, rows_v, sem):
        wid = lax.axis_index("s") * NC + lax.axis_index("c")
        base = wid * PER_W
        pltpu.sync_copy(idx_hbm.at[wid], idx_v)

        def round_body(r, carry):
            handles = []
            for g in range(GROUP):
                h = pltpu.async_copy(
                    table_hbm.at[idx_v.at[r * GROUP + g]],
                    rows_v.at[pl.ds(g * CHUNK, CHUNK)],
                    sem,
                )
                handles.append(h)
            for h in handles:
                h.wait()
            pltpu.sync_copy(rows_v, out_hbm.at[pl.ds(base + r * ROWS, ROWS)])
            return carry

        lax.fori_loop(0, ROUNDS, round_body, 0)

    return k(idx, table)


BM = 4096


def _mm_body(g_ref, w_ref, o_ref):
    o_ref[...] = lax.dot_general(
        g_ref[...], w_ref[...],
        (((1,), (1,)), ((), ())),
        preferred_element_type=jnp.float32,
    )


def _tc_project(gathered, e2):
    return pl.pallas_call(
        _mm_body,
        grid=(N // BM,),
        in_specs=[
            pl.BlockSpec((BM, D), lambda i: (i, 0)),
            pl.BlockSpec((D, D), lambda i: (0, 0)),
        ],
        out_specs=pl.BlockSpec((BM, D), lambda i: (i, 0)),
        out_shape=jax.ShapeDtypeStruct((N, D), jnp.float32),
    )(gathered, e2)


def kernel(input_ids, embedding_matrix_1, embedding_matrix_2):
    idx = input_ids.reshape(NW, N_CHUNKS, CHUNK).astype(jnp.int32)
    gathered = _sc_gather(idx, embedding_matrix_1)
    out = _tc_project(gathered, embedding_matrix_2)
    return out.reshape(B, L, D)


# speedup vs baseline: 1.5859x; 1.3563x over previous
import functools

import jax
import jax.numpy as jnp
from jax import lax
from jax.experimental import pallas as pl
from jax.experimental.pallas import tpu as pltpu
from jax.experimental.pallas import tpu_sc as plsc

NUM_EMB = 1000000
D = 64
B, L = 16384, 50
N = B * L

NC, NS = 2, 16
NW = NC * NS
PER_W = N // NW
CHUNK = 128
GROUP = 8
ROWS = CHUNK * GROUP
ROUNDS = PER_W // ROWS
N_CHUNKS = PER_W // CHUNK


def _sc_gather(idx, table):
    mesh = plsc.VectorSubcoreMesh(core_axis_name="c", subcore_axis_name="s")

    @functools.partial(
        pl.kernel,
        mesh=mesh,
        out_type=jax.ShapeDtypeStruct((N, D), jnp.float32),
        compiler_params=pltpu.CompilerParams(use_tc_tiling_on_sc=False),
        scratch_types=[
            pltpu.VMEM((N_CHUNKS, CHUNK), jnp.int32),
            pltpu.VMEM((ROWS, D), jnp.float32),
            pltpu.SemaphoreType.DMA,
        ],
    )
    def k(idx_hbm, table_hbm, out_hbm, idx_v, rows_v, sem):
        wid = lax.axis_index("s") * NC + lax.axis_index("c")
        base = wid * PER_W
        pltpu.sync_copy(idx_hbm.at[wid], idx_v)

        def round_body(r, carry):
            handles = []
            for g in range(GROUP):
                h = pltpu.async_copy(
                    table_hbm.at[idx_v.at[r * GROUP + g]],
                    rows_v.at[pl.ds(g * CHUNK, CHUNK)],
                    sem,
                )
                handles.append(h)
            for h in handles:
                h.wait()
            pltpu.sync_copy(rows_v, out_hbm.at[pl.ds(base + r * ROWS, ROWS)])
            return carry

        lax.fori_loop(0, ROUNDS, round_body, 0)

    return k(idx, table)


BN = 2048


def _mm_body(x_ref, w_ref, o_ref):
    y = lax.dot_general(
        w_ref[...], x_ref[...],
        (((1,), (1,)), ((), ())),
        preferred_element_type=jnp.float32,
    )
    o_ref[...] = y.reshape(1, D, BN)


def _tc_project(gathered_t, w):
    return pl.pallas_call(
        _mm_body,
        grid=(L, B // BN),
        in_specs=[
            pl.BlockSpec((BN, D), lambda l, i: (l * (B // BN) + i, 0)),
            pl.BlockSpec((D, D), lambda l, i: (0, 0)),
        ],
        out_specs=pl.BlockSpec((1, D, BN), lambda l, i: (l, 0, i)),
        out_shape=jax.ShapeDtypeStruct((L, D, B), jnp.float32),
    )(gathered_t, w)


def kernel(input_ids, embedding_matrix_1, embedding_matrix_2):
    idx = input_ids.T.reshape(NW, N_CHUNKS, CHUNK).astype(jnp.int32)
    gathered_t = _sc_gather(idx, embedding_matrix_1)
    out3 = _tc_project(gathered_t, embedding_matrix_2)
    return out3.transpose(2, 0, 1)


# speedup vs baseline: 2.0203x; 1.2739x over previous
import functools

import jax
import jax.numpy as jnp
from jax import lax
from jax.experimental import pallas as pl
from jax.experimental.pallas import tpu as pltpu
from jax.experimental.pallas import tpu_sc as plsc

NUM_EMB = 1000000
D = 64
B, L = 16384, 50
N = B * L

NC, NS = 2, 16
NW = NC * NS
PER_W = N // NW
CHUNK = 128
GROUP = 8
ROWS = CHUNK * GROUP
ROUNDS = PER_W // ROWS
N_CHUNKS = PER_W // CHUNK


def _sc_gather(idx, table):
    mesh = plsc.VectorSubcoreMesh(core_axis_name="c", subcore_axis_name="s")

    @functools.partial(
        pl.kernel,
        mesh=mesh,
        out_type=jax.ShapeDtypeStruct((N, 2 * D), jnp.float32),
        compiler_params=pltpu.CompilerParams(use_tc_tiling_on_sc=False),
        scratch_types=[
            pltpu.VMEM((N_CHUNKS, CHUNK), jnp.int32),
            pltpu.VMEM((ROWS, D), jnp.float32),
            pltpu.SemaphoreType.DMA,
        ],
    )
    def k(idx_hbm, table_hbm, out_hbm, idx_v, rows_v, sem):
        wid = lax.axis_index("s") * NC + lax.axis_index("c")
        base = wid * PER_W
        pltpu.sync_copy(idx_hbm.at[wid], idx_v)

        def round_body(r, carry):
            handles = []
            for g in range(GROUP):
                h = pltpu.async_copy(
                    table_hbm.at[idx_v.at[r * GROUP + g]],
                    rows_v.at[pl.ds(g * CHUNK, CHUNK)],
                    sem,
                )
                handles.append(h)
            for h in handles:
                h.wait()
            pltpu.sync_copy(
                rows_v,
                out_hbm.at[pl.ds(base + r * ROWS, ROWS), pl.ds(0, D)],
            )
            return carry

        lax.fori_loop(0, ROUNDS, round_body, 0)

    return k(idx, table)


BN = 2048


def _mm_body(x_ref, w_ref, o_ref):
    x = x_ref[:, 0:D]
    y = lax.dot_general(
        w_ref[...], x,
        (((1,), (1,)), ((), ())),
        preferred_element_type=jnp.float32,
    )
    o_ref[...] = y.reshape(1, D, BN)


def _tc_project(gathered_t, w):
    return pl.pallas_call(
        _mm_body,
        grid=(L, B // BN),
        in_specs=[
            pl.BlockSpec((BN, 2 * D), lambda l, i: (l * (B // BN) + i, 0)),
            pl.BlockSpec((D, D), lambda l, i: (0, 0)),
        ],
        out_specs=pl.BlockSpec((1, D, BN), lambda l, i: (l, 0, i)),
        out_shape=jax.ShapeDtypeStruct((L, D, B), jnp.float32),
    )(gathered_t, w)


def kernel(input_ids, embedding_matrix_1, embedding_matrix_2):
    idx = input_ids.T.reshape(NW, N_CHUNKS, CHUNK).astype(jnp.int32)
    gathered_t = _sc_gather(idx, embedding_matrix_1)
    out3 = _tc_project(gathered_t, embedding_matrix_2)
    return out3.transpose(2, 0, 1)
